# baseline (speedup 1.0000x reference)
import functools

import jax
import jax.numpy as jnp
from jax import lax
from jax.experimental import pallas as pl
from jax.experimental.pallas import tpu as pltpu
from jax.experimental.pallas import tpu_sc as plsc

L = 16
N_VALID = 68
N_PAD = 80
N_CHUNKS = N_PAD // L
N_ROWS = 100000

_mesh = plsc.VectorSubcoreMesh(
    core_axis_name="c", subcore_axis_name="s", num_cores=1, num_subcores=1
)


@functools.partial(
    pl.kernel,
    mesh=_mesh,
    compiler_params=pltpu.CompilerParams(skip_device_barrier=True),
    out_type=jax.ShapeDtypeStruct((1,), jnp.float32),
    scratch_types=[
        pltpu.VMEM((N_PAD,), jnp.int32),
        pltpu.VMEM((N_PAD,), jnp.int32),
        pltpu.VMEM((N_PAD,), jnp.float32),
        pltpu.VMEM((N_PAD,), jnp.float32),
        pltpu.VMEM((L,), jnp.float32),
        pltpu.SemaphoreType.DMA,
    ],
)
def _sc_sumsq(points_hbm, lm_hbm, out_hbm, ix_v, iy_v, vx_v, vy_v,
              out_v, sem):
    wid = lax.axis_index("s") * 2 + lax.axis_index("c")

    @pl.when(wid == 0)
    def _():
        pltpu.sync_copy(lm_hbm, ix_v.at[pl.ds(0, N_VALID)])
        lane = lax.broadcasted_iota(jnp.int32, (L,), 0)
        for j in range(N_CHUNKS):
            sl = pl.ds(j * L, L)
            idx = ix_v[sl]
            if (j + 1) * L > N_VALID:
                idx = jnp.where(j * L + lane < N_VALID, idx, 0)
                ix_v[sl] = idx
            iy_v[sl] = idx + N_ROWS
        cx = pltpu.async_copy(points_hbm.at[ix_v], vx_v, sem)
        cy = pltpu.async_copy(points_hbm.at[iy_v], vy_v, sem)
        cx.wait()
        cy.wait()

        acc = jnp.zeros((L,), jnp.float32)
        for j in range(N_CHUNKS):
            sl = pl.ds(j * L, L)
            x = vx_v[sl]
            y = vy_v[sl]
            s = x * x + y * y
            if (j + 1) * L > N_VALID:
                s = jnp.where(j * L + lane < N_VALID, s, 0.0)
            acc = acc + s
        dnums = lax.GatherDimensionNumbers(
            offset_dims=(), collapsed_slice_dims=(0,), start_index_map=(0,)
        )
        for sh in (8, 4, 2, 1):
            peer = lax.gather(
                acc, (lane ^ sh)[:, None], dnums, slice_sizes=(1,),
                mode=lax.GatherScatterMode.PROMISE_IN_BOUNDS,
            )
            acc = acc + peer
        t = acc
        y = 0.5 * (t + 1.0)
        for _ in range(16):
            y = 0.5 * (y + t / y)
        r = jnp.where(t > 0.0, y, 0.0)
        out_v[...] = r
        pltpu.sync_copy(out_v.at[pl.ds(0, 1)], out_hbm)


def kernel(pointsUV, landmarks):
    pts_lin = jnp.ravel(pointsUV.T)
    out = _sc_sumsq(pts_lin, landmarks.astype(jnp.int32))
    return out.reshape(())


# speedup vs baseline: 1.0965x; 1.0965x over previous
import functools

import jax
import jax.numpy as jnp
from jax import lax
from jax.experimental import pallas as pl
from jax.experimental.pallas import tpu as pltpu
from jax.experimental.pallas import tpu_sc as plsc

L = 16
N_VALID = 68
N_PAD = 80
N_CHUNKS = N_PAD // L
N_ROWS = 100000

_mesh = plsc.VectorSubcoreMesh(
    core_axis_name="c", subcore_axis_name="s", num_cores=1, num_subcores=1
)


@functools.partial(
    pl.kernel,
    mesh=_mesh,
    out_type=jax.ShapeDtypeStruct((1,), jnp.float32),
    scratch_types=[
        pltpu.VMEM((N_PAD,), jnp.int32),
        pltpu.VMEM((N_PAD,), jnp.int32),
        pltpu.VMEM((N_PAD,), jnp.float32),
        pltpu.VMEM((N_PAD,), jnp.float32),
        pltpu.VMEM((L,), jnp.float32),
        pltpu.SemaphoreType.DMA,
    ],
)
def _sc_sumsq(points_hbm, lm_hbm, out_hbm, lm_v, iy_v, vx_v, vy_v,
              out_v, sem):
    wid = lax.axis_index("s") * 2 + lax.axis_index("c")

    @pl.when(wid == 0)
    def _():
        pltpu.sync_copy(lm_hbm, lm_v.at[pl.ds(0, N_VALID)])
        cx = pltpu.async_copy(
            points_hbm.at[lm_v.at[pl.ds(0, N_VALID)]],
            vx_v.at[pl.ds(0, N_VALID)], sem,
        )
        lane = lax.broadcasted_iota(jnp.int32, (L,), 0)
        for j in range(N_CHUNKS):
            sl = pl.ds(j * L, L)
            iy_v[sl] = lm_v[sl] + N_ROWS
        cy = pltpu.async_copy(
            points_hbm.at[iy_v.at[pl.ds(0, N_VALID)]],
            vy_v.at[pl.ds(0, N_VALID)], sem,
        )
        cx.wait()
        cy.wait()

        acc = jnp.zeros((L,), jnp.float32)
        for j in range(N_CHUNKS):
            sl = pl.ds(j * L, L)
            x = vx_v[sl]
            y = vy_v[sl]
            s = x * x + y * y
            if (j + 1) * L > N_VALID:
                s = jnp.where(j * L + lane < N_VALID, s, 0.0)
            acc = acc + s
        dnums = lax.GatherDimensionNumbers(
            offset_dims=(), collapsed_slice_dims=(0,), start_index_map=(0,)
        )
        for sh in (8, 4, 2, 1):
            peer = lax.gather(
                acc, (lane ^ sh)[:, None], dnums, slice_sizes=(1,),
                mode=lax.GatherScatterMode.PROMISE_IN_BOUNDS,
            )
            acc = acc + peer
        t = acc
        y = 0.5 * (t + 1.0)
        for _ in range(16):
            y = 0.5 * (y + t / y)
        r = jnp.where(t > 0.0, y, 0.0)
        out_v[...] = r
        pltpu.sync_copy(out_v.at[pl.ds(0, 1)], out_hbm)


def kernel(pointsUV, landmarks):
    pts_lin = jnp.ravel(pointsUV.T)
    out = _sc_sumsq(pts_lin, landmarks.astype(jnp.int32))
    return out.reshape(())
